# baseline (speedup 1.0000x reference)
import functools

import jax
import jax.numpy as jnp
from jax import lax
from jax.experimental import pallas as pl
from jax.experimental.pallas import tpu as pltpu
from jax.experimental.pallas import tpu_sc as plsc

_ROWS = 1000

_NC = 2
_NS = 16
_LANES = 16


def _seg_sum_body(npad, d, e2, table, srcr, dstr, z, out2,
                  eb_src, eb_dst, wl_src, wl_dst, ret_src, ret_dst,
                  gb, sb, rows, accb, seen, zbuf, gsem):
    c = lax.axis_index("c")
    s = lax.axis_index("s")
    b = rows.shape[0]
    eb = eb_src.shape[0]
    zr = zbuf.shape[0]
    rpt = npad // _NS
    tile_lo = s * rpt
    dump = npad - 1
    e2h = e2 // _NC
    lane = lax.iota(jnp.int32, _LANES)

    pltpu.sync_copy(z, zbuf)
    for k in range(rpt // zr):
        pltpu.sync_copy(zbuf, out2.at[c, pl.ds(s * rpt + k * zr, zr)])

    def zseen(k, _):
        seen[pl.ds(k * _LANES, _LANES)] = jnp.zeros((_LANES,), jnp.int32)
        return 0

    lax.fori_loop(0, seen.shape[0] // _LANES, zseen, 0)

    plsc.subcore_barrier()

    def process_batch(cnt, bid):
        p0 = cnt - b
        bid = bid + 1
        base = bid * _LANES
        for k in range(ret_src.shape[0] // _LANES):
            ret_src[pl.ds(k * _LANES, _LANES)] = jnp.zeros(
                (_LANES,), jnp.int32)
            ret_dst[pl.ds(k * _LANES, _LANES)] = jnp.full(
                (_LANES,), dump, jnp.int32)
        retc = jnp.int32(0)
        for j in range(b // _LANES):
            sg = wl_src[pl.ds(p0 + j * _LANES, _LANES)]
            dg = wl_dst[pl.ds(p0 + j * _LANES, _LANES)]
            gb[pl.ds(j * _LANES, _LANES)] = sg
            valid = dg != dump
            idxs = jnp.where(valid, dg - tile_lo, rpt + lane)
            g0 = plsc.load_gather(seen, [idxs])
            tagv = base + lane
            cross = g0 >= base
            plsc.store_scatter(seen, [idxs], tagv)
            g1 = plsc.load_gather(seen, [idxs])
            win = valid & (~cross) & (g1 == tagv)
            defer = valid & (~win)
            sb[pl.ds(j * _LANES, _LANES)] = jnp.where(win, dg, dump)
            mi = defer.astype(jnp.int32)
            ps = plsc.cumsum(mi)
            pos = jnp.where(defer, retc + ps - 1, ret_src.shape[0] - _LANES)
            plsc.store_scatter(ret_src, [pos], sg)
            plsc.store_scatter(ret_dst, [pos], dg)
            retc = retc + ps[15]
        pltpu.async_copy(table.at[gb], rows, gsem).wait()
        pltpu.async_copy(out2.at[c].at[sb], accb, gsem).wait()

        def addrow(e, _):
            for k in range(d // _LANES):
                accb[e, pl.ds(k * _LANES, _LANES)] = (
                    accb[e, pl.ds(k * _LANES, _LANES)]
                    + rows[e, pl.ds(k * _LANES, _LANES)])
            return 0

        lax.fori_loop(0, b, addrow, 0)
        pltpu.sync_copy(accb, out2.at[c].at[sb])
        nr16 = (retc + _LANES - 1) // _LANES

        def app(k, _):
            wl_src[pl.ds(p0 + k * _LANES, _LANES)] = ret_src[
                pl.ds(k * _LANES, _LANES)]
            wl_dst[pl.ds(p0 + k * _LANES, _LANES)] = ret_dst[
                pl.ds(k * _LANES, _LANES)]
            return 0

        lax.fori_loop(0, nr16, app, 0)
        return p0 + nr16 * _LANES, bid

    def block_body(bi, carry):
        cnt, bid = carry
        off = pl.multiple_of(c * e2h + bi * eb, 8)
        pltpu.sync_copy(srcr.at[pl.ds(off, eb)], eb_src)
        pltpu.sync_copy(dstr.at[pl.ds(off, eb)], eb_dst)

        def chunk(i, cnt):
            sg = eb_src[pl.ds(i * _LANES, _LANES)]
            dg = eb_dst[pl.ds(i * _LANES, _LANES)]
            dl = dg - tile_lo
            m = (dl >= 0) & (dl < rpt)
            mi = m.astype(jnp.int32)
            ps = plsc.cumsum(mi)
            pos = jnp.where(m, cnt + ps - 1, wl_src.shape[0] - _LANES)
            plsc.store_scatter(wl_src, [pos], sg)
            plsc.store_scatter(wl_dst, [pos], dg)
            return cnt + ps[15]

        cnt = lax.fori_loop(0, eb // _LANES, chunk, cnt)

        padpos = cnt + lane
        plsc.store_scatter(wl_src, [padpos], jnp.zeros((_LANES,), jnp.int32))
        plsc.store_scatter(wl_dst, [padpos], jnp.full((_LANES,), dump,
                                                      jnp.int32))
        cnt = ((cnt + _LANES - 1) // _LANES) * _LANES

        def drain(carry):
            return process_batch(*carry)

        cnt, bid = lax.while_loop(lambda cb: cb[0] >= b, drain, (cnt, bid))
        return cnt, bid

    cnt, bid = lax.fori_loop(0, e2h // eb, block_body,
                             (jnp.int32(0), jnp.int32(0)))

    def tail(carry):
        cnt, bid = carry

        def pad(k, _):
            wl_src[pl.ds(cnt + k * _LANES, _LANES)] = jnp.zeros(
                (_LANES,), jnp.int32)
            wl_dst[pl.ds(cnt + k * _LANES, _LANES)] = jnp.full(
                (_LANES,), dump, jnp.int32)
            return 0

        lax.fori_loop(0, (b - cnt) // _LANES, pad, 0)
        return process_batch(b, bid)

    cnt, bid = lax.while_loop(lambda cb: cb[0] > 0, tail, (cnt, bid))

    plsc.subcore_barrier()


def _seg_sum_sc(table, src2, dst2, npad):
    _, dd = table.shape
    e2 = src2.shape[0]
    b = 128 if dd <= 256 else 64
    eb = 1024
    assert e2 % (_NC * eb) == 0
    rpt = npad // _NS
    z = jnp.zeros((64, dd), jnp.float32)
    mesh = plsc.VectorSubcoreMesh(core_axis_name="c", subcore_axis_name="s",
                                  num_cores=_NC, num_subcores=_NS)
    body = functools.partial(_seg_sum_body, npad, dd, e2)
    out2 = pl.kernel(
        body,
        out_type=jax.ShapeDtypeStruct((_NC, npad, dd), jnp.float32),
        mesh=mesh,
        compiler_params=pltpu.CompilerParams(needs_layout_passes=False),
        scratch_types=[
            pltpu.VMEM((eb,), jnp.int32),
            pltpu.VMEM((eb,), jnp.int32),
            pltpu.VMEM((2048,), jnp.int32),
            pltpu.VMEM((2048,), jnp.int32),
            pltpu.VMEM((176,), jnp.int32),
            pltpu.VMEM((176,), jnp.int32),
            pltpu.VMEM((b,), jnp.int32),
            pltpu.VMEM((b,), jnp.int32),
            pltpu.VMEM((b, dd), jnp.float32),
            pltpu.VMEM((b, dd), jnp.float32),
            pltpu.VMEM((rpt + _LANES,), jnp.int32),
            pltpu.VMEM((64, dd), jnp.float32),
            pltpu.SemaphoreType.DMA,
        ],
    )(table, src2, dst2, z)
    return out2


def _mlp_body(scale_ref, x_ref, agg0_ref, agg1_ref, wa_ref, ba_ref, wb_ref,
              bb_ref, out_ref, *, final_relu):
    t = scale_ref[0] * x_ref[...] + agg0_ref[0] + agg1_ref[0]
    h = jnp.dot(t, wa_ref[...], preferred_element_type=jnp.float32)
    h = jnp.maximum(h + ba_ref[...], 0.0)
    o = jnp.dot(h, wb_ref[...], preferred_element_type=jnp.float32)
    o = o + bb_ref[...]
    if final_relu:
        o = jnp.maximum(o, 0.0)
    out_ref[...] = o


def _gin_mlp(x, agg2, scale, Wa, ba, Wb, bb, final_relu, rows=2560):
    n, din = x.shape
    dmid = Wa.shape[1]
    dout = Wb.shape[1]
    _ROWS = rows
    grid = (n // _ROWS,)
    return pl.pallas_call(
        functools.partial(_mlp_body, final_relu=final_relu),
        grid=grid,
        in_specs=[
            pl.BlockSpec(memory_space=pltpu.SMEM),
            pl.BlockSpec((_ROWS, din), lambda i: (i, 0)),
            pl.BlockSpec((1, _ROWS, din), lambda i: (0, i, 0)),
            pl.BlockSpec((1, _ROWS, din), lambda i: (1, i, 0)),
            pl.BlockSpec((din, dmid), lambda i: (0, 0)),
            pl.BlockSpec((1, dmid), lambda i: (0, 0)),
            pl.BlockSpec((dmid, dout), lambda i: (0, 0)),
            pl.BlockSpec((1, dout), lambda i: (0, 0)),
        ],
        out_specs=pl.BlockSpec((_ROWS, dout), lambda i: (i, 0)),
        out_shape=jax.ShapeDtypeStruct((n, dout), jnp.float32),
    )(scale, x, agg2, agg2, Wa, ba.reshape(1, -1), Wb, bb.reshape(1, -1))


def _cluster_body(temp_ref, h_ref, mu_ref, out_ref):
    h = h_ref[...]
    nrm = jnp.sqrt(jnp.sum(h * h, axis=1, keepdims=True))
    data = h / (nrm + 1e-8)
    temp = temp_ref[0]

    def it(_, mu):
        dist = lax.dot_general(data, mu, (((1,), (1,)), ((), ())),
                               preferred_element_type=jnp.float32)
        logits = temp * dist
        m = jnp.max(logits, axis=1, keepdims=True)
        e = jnp.exp(logits - m)
        r = e / jnp.sum(e, axis=1, keepdims=True)
        cr = jnp.sum(r, axis=0)
        cm = lax.dot_general(r, data, (((0,), (0,)), ((), ())),
                             preferred_element_type=jnp.float32)
        return cm / (cr[:, None] + 1e-8)

    out_ref[...] = lax.fori_loop(0, 11, it, mu_ref[...])


def _cluster_mu(h1, mu_init, temp):
    k, nh = mu_init.shape
    n = h1.shape[0]
    return pl.pallas_call(
        _cluster_body,
        in_specs=[
            pl.BlockSpec(memory_space=pltpu.SMEM),
            pl.BlockSpec((n, nh), lambda: (0, 0)),
            pl.BlockSpec((k, nh), lambda: (0, 0)),
        ],
        out_specs=pl.BlockSpec((k, nh), lambda: (0, 0)),
        out_shape=jax.ShapeDtypeStruct((k, nh), jnp.float32),
    )(temp, h1, mu_init)


def _fin1_body(temp_ref, h1_ref, h2_ref, mu_ref, mskT_ref, sb1_ref, sb2_ref,
               s1_ref, s2_ref, csum_ref, msum_ref):
    h1 = h1_ref[...]
    h2 = h2_ref[...]
    mu = mu_ref[...]
    nrm = jnp.sqrt(jnp.sum(h1 * h1, axis=1, keepdims=True))
    data = h1 / (nrm + 1e-8)
    dist = lax.dot_general(data, mu, (((1,), (1,)), ((), ())),
                           preferred_element_type=jnp.float32)
    logits = temp_ref[0] * dist
    m = jnp.max(logits, axis=1, keepdims=True)
    e = jnp.exp(logits - m)
    s = e / jnp.sum(e, axis=1, keepdims=True)
    zt = jnp.dot(s, mu, preferred_element_type=jnp.float32)
    c2 = jax.nn.sigmoid(zt)
    s1_ref[...] = jnp.sum(h1 * c2, axis=1, keepdims=True) + sb1_ref[...]
    s2_ref[...] = jnp.sum(h2 * c2, axis=1, keepdims=True) + sb2_ref[...]

    mskT = mskT_ref[...]

    @pl.when(pl.program_id(0) == 0)
    def _():
        csum_ref[...] = jnp.zeros_like(csum_ref)
        msum_ref[...] = jnp.zeros_like(msum_ref)

    csum_ref[...] += jnp.sum(h1 * mskT, axis=0, keepdims=True)
    msum_ref[...] += jnp.sum(mskT, keepdims=True)


def _finals1(h1, h2, mu, mskT, sb1T, sb2T, temp):
    n, nh = h1.shape
    k = mu.shape[0]
    grid = (n // _ROWS,)
    return pl.pallas_call(
        _fin1_body,
        grid=grid,
        in_specs=[
            pl.BlockSpec(memory_space=pltpu.SMEM),
            pl.BlockSpec((_ROWS, nh), lambda i: (i, 0)),
            pl.BlockSpec((_ROWS, nh), lambda i: (i, 0)),
            pl.BlockSpec((k, nh), lambda i: (0, 0)),
            pl.BlockSpec((_ROWS, 1), lambda i: (i, 0)),
            pl.BlockSpec((_ROWS, 1), lambda i: (i, 0)),
            pl.BlockSpec((_ROWS, 1), lambda i: (i, 0)),
        ],
        out_specs=[
            pl.BlockSpec((_ROWS, 1), lambda i: (i, 0)),
            pl.BlockSpec((_ROWS, 1), lambda i: (i, 0)),
            pl.BlockSpec((1, nh), lambda i: (0, 0)),
            pl.BlockSpec((1, 1), lambda i: (0, 0)),
        ],
        out_shape=[
            jax.ShapeDtypeStruct((n, 1), jnp.float32),
            jax.ShapeDtypeStruct((n, 1), jnp.float32),
            jax.ShapeDtypeStruct((1, nh), jnp.float32),
            jax.ShapeDtypeStruct((1, 1), jnp.float32),
        ],
    )(temp, h1, h2, mu, mskT, sb1T, sb2T)


def _fin2_body(bd_ref, csum_ref, msum_ref, wd_ref, h1_ref, h2_ref, sb1_ref,
               sb2_ref, o1_ref, o2_ref):
    c = jax.nn.sigmoid(csum_ref[...] / msum_ref[...])
    q = lax.dot_general(wd_ref[...], c, (((1,), (1,)), ((), ())),
                        preferred_element_type=jnp.float32)
    bd = bd_ref[0]
    o1_ref[...] = jnp.dot(h1_ref[...], q,
                          preferred_element_type=jnp.float32) + bd + sb1_ref[...]
    o2_ref[...] = jnp.dot(h2_ref[...], q,
                          preferred_element_type=jnp.float32) + bd + sb2_ref[...]


def _finals2(h1, h2, csum, msum, Wd, bd, sb1T, sb2T):
    n, nh = h1.shape
    grid = (n // _ROWS,)
    return pl.pallas_call(
        _fin2_body,
        grid=grid,
        in_specs=[
            pl.BlockSpec(memory_space=pltpu.SMEM),
            pl.BlockSpec((1, nh), lambda i: (0, 0)),
            pl.BlockSpec((1, 1), lambda i: (0, 0)),
            pl.BlockSpec((nh, nh), lambda i: (0, 0)),
            pl.BlockSpec((_ROWS, nh), lambda i: (i, 0)),
            pl.BlockSpec((_ROWS, nh), lambda i: (i, 0)),
            pl.BlockSpec((_ROWS, 1), lambda i: (i, 0)),
            pl.BlockSpec((_ROWS, 1), lambda i: (i, 0)),
        ],
        out_specs=[
            pl.BlockSpec((_ROWS, 1), lambda i: (i, 0)),
            pl.BlockSpec((_ROWS, 1), lambda i: (i, 0)),
        ],
        out_shape=[
            jax.ShapeDtypeStruct((n, 1), jnp.float32),
            jax.ShapeDtypeStruct((n, 1), jnp.float32),
        ],
    )(bd, csum, msum, Wd, h1, h2, sb1T, sb2T)


_NPAD = 20480


def kernel(seq1, seq2, g, sparse, msk, samp_bias1, samp_bias2, cluster_temp,
           W1a, b1a, W1b, b1b, eps1, W2a, b2a, W2b, b2b, eps2, Wd, bd,
           mu_init):
    n = seq1.shape[1]
    src = g[0]
    dst = g[1]
    x1 = seq1[0]
    x2 = seq2[0]
    din = x1.shape[1]

    scale1 = (1.0 + eps1).reshape(1).astype(jnp.float32)
    scale2 = (1.0 + eps2).reshape(1).astype(jnp.float32)
    temp = jnp.asarray(cluster_temp, dtype=jnp.float32).reshape(1)
    bdv = jnp.asarray(bd, dtype=jnp.float32).reshape(1)

    xcat = jnp.concatenate(
        [x1, x2, jnp.zeros((_NPAD - 2 * n, din), jnp.float32)], axis=0)
    e = src.shape[0]
    e2p = ((2 * e + 4095) // 4096) * 4096
    npadrow = jnp.full((e2p - 2 * e,), _NPAD - 64, jnp.int32)
    src2 = jnp.concatenate([src, src + n, jnp.zeros_like(npadrow)], axis=0)
    dst2 = jnp.concatenate([dst, dst + n, npadrow], axis=0)

    agg0 = _seg_sum_sc(xcat, src2, dst2, _NPAD)
    hcat = _gin_mlp(xcat, agg0, scale1, W1a, b1a, W1b, b1b, final_relu=True)

    agg1 = _seg_sum_sc(hcat, src2, dst2, _NPAD)
    ocat = _gin_mlp(hcat, agg1, scale2, W2a, b2a, W2b, b2b, final_relu=False)
    h1 = ocat[:n]
    h2 = ocat[n:2 * n]

    mu = _cluster_mu(h1, mu_init, temp)

    mskT = msk.reshape(n, 1)
    sb1T = samp_bias1.reshape(n, 1)
    sb2T = samp_bias2.reshape(n, 1)
    s1, s2, csum, msum = _finals1(h1, h2, mu, mskT, sb1T, sb2T, temp)
    o1, o2 = _finals2(h1, h2, csum, msum, Wd, bdv, sb1T, sb2T)

    ret = jnp.concatenate([o1.reshape(1, n), o2.reshape(1, n)], axis=1)
    ret2 = jnp.concatenate([s1.reshape(1, n), s2.reshape(1, n)], axis=1)
    return (ret, ret2)
